# Initial kernel scaffold; baseline (speedup 1.0000x reference)
import jax
import jax.numpy as jnp
from jax.experimental import pallas as pl


def kernel(x, W_in, U_in, C_in, B_in, W_h0, U_h0, C_h0, B_h0, W_h1, U_h1, C_h1, B_h1, W_out, U_out, C_out, B_out, edge_index):
    raise NotImplementedError("write your pallas kernel here")


# speedup vs baseline: 8.1223x; 8.1223x over previous
import functools

import jax
import jax.numpy as jnp
from jax import lax
from jax.experimental import pallas as pl
from jax.experimental.pallas import tpu as pltpu
from jax.experimental.pallas import tpu_sc as plsc

N_NODES = 10000
D_FEAT = 128
N_EDGES = 320000

NUM_CORES = 2
NUM_SUBCORES = 16
NUM_WORKERS = NUM_CORES * NUM_SUBCORES

CHUNK = 128
EDGES_PER_WORKER = N_EDGES // NUM_WORKERS
NCHUNK = -(-EDGES_PER_WORKER // CHUNK)
EPW_PAD = NCHUNK * CHUNK
E_PAD = EPW_PAD * NUM_WORKERS

TRASH = N_NODES
ACC_ROWS = 10112
ZROWS = ACC_ROWS // NUM_SUBCORES

ROW_BLOCK = 400
DEG_LANES = 128

_MESH = plsc.VectorSubcoreMesh(core_axis_name="c", subcore_axis_name="s")


@functools.partial(
    pl.kernel, mesh=_MESH,
    out_type=jax.ShapeDtypeStruct((NUM_CORES, ACC_ROWS, D_FEAT), jnp.float32),
    scratch_types=[
        pltpu.VMEM((NCHUNK, CHUNK), jnp.int32),
        pltpu.VMEM((NCHUNK, CHUNK), jnp.int32),
        pltpu.VMEM((CHUNK, D_FEAT), jnp.float32),
        pltpu.VMEM_SHARED((ACC_ROWS, D_FEAT), jnp.float32),
        pltpu.SemaphoreType.DMA,
    ])
def _sc_agg(h_hbm, srci_hbm, dsti_hbm, zeros_hbm, outa_hbm,
            src_v, dst_v, rows_v, acc_sh, sem):
    c = lax.axis_index("c")
    s = lax.axis_index("s")
    wid = s * NUM_CORES + c

    pltpu.sync_copy(zeros_hbm, acc_sh.at[pl.ds(s * ZROWS, ZROWS)])
    pltpu.sync_copy(srci_hbm.at[wid], src_v)
    pltpu.sync_copy(dsti_hbm.at[wid], dst_v)
    plsc.subcore_barrier()

    def body(j, carry):
        pltpu.async_copy(h_hbm.at[src_v.at[j]], rows_v, sem).wait()
        pltpu.sync_copy(rows_v, acc_sh.at[dst_v.at[j]], add=True)
        return carry

    lax.fori_loop(0, NCHUNK, body, 0)
    plsc.subcore_barrier()

    r0 = s * ZROWS
    pltpu.sync_copy(acc_sh.at[pl.ds(r0, ZROWS)],
                    outa_hbm.at[c, pl.ds(r0, ZROWS)])


@functools.partial(
    pl.kernel, mesh=_MESH,
    out_type=jax.ShapeDtypeStruct((NUM_CORES, ACC_ROWS, DEG_LANES),
                                  jnp.float32),
    scratch_types=[
        pltpu.VMEM((NCHUNK, CHUNK), jnp.int32),
        pltpu.VMEM((CHUNK, DEG_LANES), jnp.float32),
        pltpu.VMEM_SHARED((ACC_ROWS, DEG_LANES), jnp.float32),
        pltpu.SemaphoreType.DMA,
    ])
def _sc_degree(dsti_hbm, zerosd_hbm, ones_hbm, outd_hbm,
               dst_v, ones_v, deg_sh, sem):
    c = lax.axis_index("c")
    s = lax.axis_index("s")
    wid = s * NUM_CORES + c

    pltpu.sync_copy(zerosd_hbm, deg_sh.at[pl.ds(s * ZROWS, ZROWS)])
    pltpu.sync_copy(dsti_hbm.at[wid], dst_v)
    pltpu.sync_copy(ones_hbm, ones_v)
    plsc.subcore_barrier()

    def body(j, carry):
        pltpu.sync_copy(ones_v, deg_sh.at[dst_v.at[j]], add=True)
        return carry

    lax.fori_loop(0, NCHUNK, body, 0)
    plsc.subcore_barrier()

    r0 = s * ZROWS
    pltpu.sync_copy(deg_sh.at[pl.ds(r0, ZROWS)],
                    outd_hbm.at[c, pl.ds(r0, ZROWS)])


def _tc_layer_body(ap_ref, h_ref, degp_ref, w_ref, b_ref, o_ref, *, act):
    a = ap_ref[0] + ap_ref[1] + h_ref[...]
    deg = degp_ref[0, :, 0:1] + degp_ref[1, :, 0:1] + 1.0
    y = jnp.dot(a, w_ref[...], preferred_element_type=jnp.float32)
    y = y / deg + b_ref[...]
    o_ref[...] = act(y)


def _make_tc_layer(out_c, act):
    grid = (N_NODES // ROW_BLOCK,)
    return pl.pallas_call(
        functools.partial(_tc_layer_body, act=act),
        grid=grid,
        in_specs=[
            pl.BlockSpec((NUM_CORES, ROW_BLOCK, D_FEAT), lambda i: (0, i, 0)),
            pl.BlockSpec((ROW_BLOCK, D_FEAT), lambda i: (i, 0)),
            pl.BlockSpec((NUM_CORES, ROW_BLOCK, DEG_LANES), lambda i: (0, i, 0)),
            pl.BlockSpec((D_FEAT, out_c), lambda i: (0, 0)),
            pl.BlockSpec((1, out_c), lambda i: (0, 0)),
        ],
        out_specs=pl.BlockSpec((ROW_BLOCK, out_c), lambda i: (i, 0)),
        out_shape=jax.ShapeDtypeStruct((N_NODES, out_c), jnp.float32),
    )


_relu = lambda y: jnp.maximum(y, 0.0)
_tc_hidden = _make_tc_layer(D_FEAT, _relu)
_tc_final = _make_tc_layer(64, jnp.tanh)


def kernel(x, W_in, U_in, C_in, B_in, W_h0, U_h0, C_h0, B_h0,
           W_h1, U_h1, C_h1, B_h1, W_out, U_out, C_out, B_out, edge_index):
    src = edge_index[0].astype(jnp.int32)
    dst = edge_index[1].astype(jnp.int32)
    dstp = jnp.where(src == dst, TRASH, dst)
    pad = E_PAD - N_EDGES
    srci = jnp.concatenate([src, jnp.zeros((pad,), jnp.int32)])
    dsti = jnp.concatenate([dstp, jnp.full((pad,), TRASH, jnp.int32)])
    srci = srci.reshape(NUM_WORKERS, NCHUNK, CHUNK)
    dsti = dsti.reshape(NUM_WORKERS, NCHUNK, CHUNK)

    zeros_h = jnp.zeros((ZROWS, D_FEAT), jnp.float32)
    zerosd_h = jnp.zeros((ZROWS, DEG_LANES), jnp.float32)
    ones_h = jnp.ones((CHUNK, DEG_LANES), jnp.float32)

    degp = _sc_degree(dsti, zerosd_h, ones_h)
    ap = _sc_agg(x, srci, dsti, zeros_h)
    h = _tc_hidden(ap, x, degp, W_in, B_in.reshape(1, -1))
    ap = _sc_agg(h, srci, dsti, zeros_h)
    h = _tc_hidden(ap, h, degp, W_h0, B_h0.reshape(1, -1))
    ap = _sc_agg(h, srci, dsti, zeros_h)
    h = _tc_hidden(ap, h, degp, W_h1, B_h1.reshape(1, -1))
    ap = _sc_agg(h, srci, dsti, zeros_h)
    return _tc_final(ap, h, degp, W_out, B_out.reshape(1, -1))
